# baseline (speedup 1.0000x reference)
import jax
import jax.numpy as jnp
from jax import lax
from jax.experimental import pallas as pl
from jax.experimental.pallas import tpu as pltpu
from jax.experimental.pallas import tpu_sc as plsc

_MOM = 0.9
_B = 16384
_D = 128
_N = 200000
_C = 100
_NC = 2
_NS = 16
_NW = _NC * _NS
_BW = _B // _NW
_HALF = _BW // 2
_SEG = _B // _NS
_HALF_N = _N // _NC
_SHARE = 6248
_UPAD = _B + _NC * 8

_GDN = lax.GatherDimensionNumbers(
    offset_dims=(), collapsed_slice_dims=(0,), start_index_map=(0,))


def _shuf(v, perm):
    return lax.gather(v, perm[:, None], _GDN, (1,),
                      mode=lax.GatherScatterMode.PROMISE_IN_BOUNDS)


_LANE = lambda: lax.broadcasted_iota(jnp.int32, (16,), 0)


def _hsum16(v):
    lane = _LANE()
    for k in (8, 4, 2, 1):
        v = v + _shuf(v, lax.bitwise_xor(lane, k))
    return v


def _rsqrt(s):
    i = lax.bitcast_convert_type(s, jnp.int32)
    i = jnp.int32(0x5F3759DF) - lax.shift_right_arithmetic(i, 1)
    y = lax.bitcast_convert_type(i, jnp.float32)
    for _ in range(3):
        y = y * (1.5 - 0.5 * s * y * y)
    return y


def _k1_body(batch_hbm, idx_hbm, pos_hbm, lat_hbm,
             upd_hbm, winpos_hbm,
             idxv, rowsv, bv, idxfull, posv, segi, segw, winner_sp, sem):
    c = lax.axis_index("c")
    s = lax.axis_index("s")
    w = c * _NS + s
    base = w * _BW

    @pl.when(jnp.logical_and(c == 0, s == 0))
    def _():
        pltpu.sync_copy(idx_hbm, idxfull)
        pltpu.sync_copy(pos_hbm, posv)
        pltpu.sync_copy(posv, winner_sp.at[idxfull])

    @pl.when(s == 1)
    def _():
        pltpu.async_copy(lat_hbm.at[pl.ds(c * _HALF_N, 8)],
                         rowsv.at[pl.ds(0, 8)], sem).wait()
        pltpu.sync_copy(rowsv.at[pl.ds(0, 8)],
                        upd_hbm.at[pl.ds(_B + c * 8, 8)])

    pltpu.sync_copy(idx_hbm.at[pl.ds(base, _BW)], idxv)
    for h in range(2):
        r0 = base + h * _HALF
        pltpu.async_copy(lat_hbm.at[idxv.at[pl.ds(h * _HALF, _HALF)]],
                         rowsv, sem).wait()
        pltpu.sync_copy(batch_hbm.at[pl.ds(r0, _HALF)], bv)

        def row(r, _):
            ss = jnp.zeros((16,), jnp.float32)
            us = []
            for k in range(8):
                m = rowsv[r, pl.ds(k * 16, 16)]
                b = bv[r, pl.ds(k * 16, 16)]
                u = m * (1.0 - _MOM) + b * _MOM
                us.append(u)
                ss = ss + u * u
            scale = _rsqrt(_hsum16(ss))
            for k in range(8):
                rowsv[r, pl.ds(k * 16, 16)] = us[k] * scale
            return 0

        lax.fori_loop(0, _HALF, row, 0)
        pltpu.sync_copy(rowsv, upd_hbm.at[pl.ds(r0, _HALF)])

    plsc.subcore_barrier()

    @pl.when(c == 0)
    def _():
        p0 = s * _SEG
        pltpu.sync_copy(idx_hbm.at[pl.ds(p0, _SEG)], segi)
        pltpu.async_copy(winner_sp.at[segi], segw, sem).wait()
        pltpu.sync_copy(segw, winpos_hbm.at[pl.ds(p0, _SEG)])


def _k2_body(lat_hbm, upd_hbm, idx_hbm, winpos_hbm, fb_hbm,
             out_hbm,
             segi, segw, cidx, cwpos, fbv, rows0, rows1, cbuf0, cbuf1,
             sem0, sem1, wsem0, wsem1):
    c = lax.axis_index("c")
    s = lax.axis_index("s")
    lo = c * _HALF_N

    r0 = lo + s * _SHARE
    _CH = 256
    ncp = _SHARE // _CH
    tail = _SHARE - ncp * _CH

    pltpu.async_copy(lat_hbm.at[pl.ds(r0, _CH)], cbuf0, sem0)
    for j in range(ncp):
        buf, sem, wsem = (cbuf0, sem0, wsem0) if j % 2 == 0 else \
                         (cbuf1, sem1, wsem1)
        nbuf, nsem, nwsem = (cbuf1, sem1, wsem1) if j % 2 == 0 else \
                            (cbuf0, sem0, wsem0)
        pltpu.make_async_copy(lat_hbm.at[pl.ds(r0 + j * _CH, _CH)],
                              buf, sem).wait()
        if j + 1 < ncp:
            if j >= 1:
                pltpu.make_async_copy(
                    nbuf, out_hbm.at[pl.ds(r0 + (j - 1) * _CH, _CH)],
                    nwsem).wait()
            pltpu.async_copy(lat_hbm.at[pl.ds(r0 + (j + 1) * _CH, _CH)],
                             nbuf, nsem)
        pltpu.async_copy(buf, out_hbm.at[pl.ds(r0 + j * _CH, _CH)], wsem)
    for j in (ncp - 2, ncp - 1):
        buf, wsem = (cbuf0, wsem0) if j % 2 == 0 else (cbuf1, wsem1)
        pltpu.make_async_copy(buf, out_hbm.at[pl.ds(r0 + j * _CH, _CH)],
                              wsem).wait()

    t0 = r0 + ncp * _CH
    pltpu.async_copy(lat_hbm.at[pl.ds(t0, tail)],
                     cbuf0.at[pl.ds(0, tail)], sem0).wait()
    pltpu.sync_copy(cbuf0.at[pl.ds(0, tail)], out_hbm.at[pl.ds(t0, tail)])

    @pl.when(s == _NS - 1)
    def _():
        r1 = lo + _NS * _SHARE
        rem = _HALF_N - _NS * _SHARE
        pltpu.async_copy(lat_hbm.at[pl.ds(r1, rem)],
                         cbuf1.at[pl.ds(0, rem)], sem1).wait()
        pltpu.sync_copy(cbuf1.at[pl.ds(0, rem)], out_hbm.at[pl.ds(r1, rem)])

    plsc.subcore_barrier()

    p0 = s * _SEG
    pltpu.sync_copy(idx_hbm.at[pl.ds(p0, _SEG)], segi)
    pltpu.sync_copy(winpos_hbm.at[pl.ds(p0, _SEG)], segw)
    pltpu.sync_copy(fb_hbm.at[c], fbv)
    zeros = jnp.zeros((16,), jnp.int32)
    jf = _shuf(fbv[pl.ds(0, 16)], zeros)
    wf = _shuf(fbv[pl.ds(0, 16)], zeros + 1)

    lane = _LANE()

    def fill(t, _):
        iv = segi[pl.ds(t * 16, 16)]
        wv = segw[pl.ds(t * 16, 16)]
        owned = jnp.logical_and(iv >= lo, iv < lo + _HALF_N)
        mn = jnp.where(owned, lane, 64)
        for k in (8, 4, 2, 1):
            mn = jnp.minimum(mn, _shuf(mn, lax.bitwise_xor(lane, k)))
        has = mn < 16
        perm = jnp.where(has, mn, 0)
        iv2 = jnp.where(has, _shuf(iv, perm), jf)
        wv2 = jnp.where(has, _shuf(wv, perm), wf)
        cidx[t // 8, pl.ds((t % 8) * 16, 16)] = jnp.where(owned, iv, iv2)
        cwpos[t // 8, pl.ds((t % 8) * 16, 16)] = jnp.where(owned, wv, wv2)
        return 0

    lax.fori_loop(0, _SEG // 16, fill, 0)

    nch = _SEG // 128
    pltpu.async_copy(upd_hbm.at[cwpos.at[0]], rows0, sem0)
    for j in range(nch):
        buf, sem = (rows0, sem0) if j % 2 == 0 else (rows1, sem1)
        nbuf, nsem = (rows1, sem1) if j % 2 == 0 else (rows0, sem0)
        pltpu.make_async_copy(upd_hbm.at[cwpos.at[j]], buf, sem).wait()
        if j + 1 < nch:
            pltpu.async_copy(upd_hbm.at[cwpos.at[j + 1]], nbuf, nsem)
        pltpu.sync_copy(buf, out_hbm.at[cidx.at[j]])


def _sc_call(inputs, body, out_type, scratch):
    mesh = plsc.VectorSubcoreMesh(core_axis_name="c", subcore_axis_name="s",
                                  num_cores=_NC, num_subcores=_NS)
    return pl.kernel(body, out_type=out_type, mesh=mesh,
                     scratch_types=scratch)(*inputs)


def _k3_body(t_ref, u_ref, cs_ref, out_ref):
    i = pl.program_id(0)
    t = t_ref[0, 0, :]
    oh = jnp.where(
        lax.broadcasted_iota(jnp.int32, (_C, t.shape[0]), 0) == t[None, :],
        1.0, 0.0)
    acc = jnp.dot(oh, u_ref[...], preferred_element_type=jnp.float32,
                  precision=lax.Precision.HIGHEST)

    @pl.when(i == 0)
    def _():
        out_ref[...] = cs_ref[...] + acc

    @pl.when(i > 0)
    def _():
        out_ref[...] = out_ref[...] + acc


def kernel(batch_samples, targets, idx, lat_memory, class_sums):
    pos = jnp.arange(_B, dtype=jnp.int32)

    fb_rows = []
    for h in range(_NC):
        in_half = jnp.logical_and(idx >= h * _HALF_N, idx < (h + 1) * _HALF_N)
        any_h = jnp.any(in_half)
        p = jnp.argmax(in_half).astype(jnp.int32)
        jf = jnp.where(any_h, idx[p], h * _HALF_N)
        fb_rows.append([jf, jnp.where(any_h, -1, _B + h * 8), p])
    fb = jnp.zeros((_NC, 16), jnp.int32)
    fb = fb.at[0, 0].set(fb_rows[0][0]).at[1, 0].set(fb_rows[1][0])

    upd, winpos = _sc_call(
        (batch_samples, idx, pos, lat_memory),
        _k1_body,
        [jax.ShapeDtypeStruct((_UPAD, _D), jnp.float32),
         jax.ShapeDtypeStruct((_B,), jnp.int32)],
        [
            pltpu.VMEM((_BW,), jnp.int32),
            pltpu.VMEM((_HALF, _D), jnp.float32),
            pltpu.VMEM((_HALF, _D), jnp.float32),
            pltpu.VMEM((_B,), jnp.int32),
            pltpu.VMEM((_B,), jnp.int32),
            pltpu.VMEM((_SEG,), jnp.int32),
            pltpu.VMEM((_SEG,), jnp.int32),
            pltpu.VMEM_SHARED((_N,), jnp.int32),
            pltpu.SemaphoreType.DMA,
        ])

    wsel = jnp.stack([
        jnp.where(fb_rows[0][1] < 0, winpos[fb_rows[0][2]], fb_rows[0][1]),
        jnp.where(fb_rows[1][1] < 0, winpos[fb_rows[1][2]], fb_rows[1][1]),
    ]).astype(jnp.int32)
    fb = fb.at[0, 1].set(wsel[0]).at[1, 1].set(wsel[1])

    new_lat = _sc_call(
        (lat_memory, upd, idx, winpos, fb),
        _k2_body,
        jax.ShapeDtypeStruct((_N, _D), jnp.float32),
        [
            pltpu.VMEM((_SEG,), jnp.int32),
            pltpu.VMEM((_SEG,), jnp.int32),
            pltpu.VMEM((_SEG // 128, 128), jnp.int32),
            pltpu.VMEM((_SEG // 128, 128), jnp.int32),
            pltpu.VMEM((16,), jnp.int32),
            pltpu.VMEM((128, _D), jnp.float32),
            pltpu.VMEM((128, _D), jnp.float32),
            pltpu.VMEM((256, _D), jnp.float32),
            pltpu.VMEM((256, _D), jnp.float32),
            pltpu.SemaphoreType.DMA,
            pltpu.SemaphoreType.DMA,
            pltpu.SemaphoreType.DMA,
            pltpu.SemaphoreType.DMA,
        ])

    tchunk = 2048
    new_cs = pl.pallas_call(
        _k3_body,
        grid=(_B // tchunk,),
        in_specs=[
            pl.BlockSpec((1, 1, tchunk), lambda i: (i, 0, 0)),
            pl.BlockSpec((tchunk, _D), lambda i: (i, 0)),
            pl.BlockSpec((_C, _D), lambda i: (0, 0)),
        ],
        out_specs=pl.BlockSpec((_C, _D), lambda i: (0, 0)),
        out_shape=jax.ShapeDtypeStruct((_C, _D), jnp.float32),
    )(targets.reshape(_B // tchunk, 1, tchunk), upd, class_sums)

    return new_lat, new_cs


# speedup vs baseline: 1.0246x; 1.0246x over previous
import jax
import jax.numpy as jnp
from jax import lax
from jax.experimental import pallas as pl
from jax.experimental.pallas import tpu as pltpu
from jax.experimental.pallas import tpu_sc as plsc

_MOM = 0.9
_B = 16384
_D = 128
_N = 200000
_C = 100
_NC = 2
_NS = 16
_NW = _NC * _NS
_BW = _B // _NW
_HALF = _BW // 2
_SEG = _B // _NS
_HALF_N = _N // _NC
_SHARE = 6248
_UPAD = _B + _NC * 8

_GDN = lax.GatherDimensionNumbers(
    offset_dims=(), collapsed_slice_dims=(0,), start_index_map=(0,))


def _shuf(v, perm):
    return lax.gather(v, perm[:, None], _GDN, (1,),
                      mode=lax.GatherScatterMode.PROMISE_IN_BOUNDS)


_LANE = lambda: lax.broadcasted_iota(jnp.int32, (16,), 0)


def _hsum16(v):
    lane = _LANE()
    for k in (8, 4, 2, 1):
        v = v + _shuf(v, lax.bitwise_xor(lane, k))
    return v


def _rsqrt(s):
    i = lax.bitcast_convert_type(s, jnp.int32)
    i = jnp.int32(0x5F3759DF) - lax.shift_right_arithmetic(i, 1)
    y = lax.bitcast_convert_type(i, jnp.float32)
    for _ in range(3):
        y = y * (1.5 - 0.5 * s * y * y)
    return y


def _k1_body(batch_hbm, idx_hbm, pos_hbm, lat_hbm,
             upd_hbm, winpos_hbm,
             idxv, rowsv, bv, rowsv1, bv1, idxfull, posv, segi, segw,
             winner_sp, sem, sem1):
    c = lax.axis_index("c")
    s = lax.axis_index("s")
    w = c * _NS + s
    base = w * _BW

    @pl.when(s == 0)
    def _():
        pltpu.sync_copy(idx_hbm, idxfull)
        pltpu.sync_copy(pos_hbm, posv)
        pltpu.sync_copy(posv, winner_sp.at[idxfull])

    @pl.when(s == 1)
    def _():
        pltpu.async_copy(lat_hbm.at[pl.ds(c * _HALF_N, 8)],
                         rowsv.at[pl.ds(0, 8)], sem).wait()
        pltpu.sync_copy(rowsv.at[pl.ds(0, 8)],
                        upd_hbm.at[pl.ds(_B + c * 8, 8)])

    pltpu.sync_copy(idx_hbm.at[pl.ds(base, _BW)], idxv)
    _Q = _BW // 4

    def fire(q, rv, bb, sm):
        pltpu.async_copy(lat_hbm.at[idxv.at[pl.ds(q * _Q, _Q)]], rv, sm)
        pltpu.async_copy(batch_hbm.at[pl.ds(base + q * _Q, _Q)], bb, sm)

    fire(0, rowsv, bv, sem)
    for q in range(4):
        rv, bb, sm = (rowsv, bv, sem) if q % 2 == 0 else (rowsv1, bv1, sem1)
        nrv, nbb, nsm = (rowsv1, bv1, sem1) if q % 2 == 0 else (rowsv, bv, sem)
        pltpu.make_async_copy(lat_hbm.at[idxv.at[pl.ds(q * _Q, _Q)]],
                              rv, sm).wait()
        pltpu.make_async_copy(batch_hbm.at[pl.ds(base + q * _Q, _Q)],
                              bb, sm).wait()
        if q + 1 < 4:
            fire(q + 1, nrv, nbb, nsm)

        def row(r, _):
            ss = jnp.zeros((16,), jnp.float32)
            us = []
            for k in range(8):
                m = rv[r, pl.ds(k * 16, 16)]
                b = bb[r, pl.ds(k * 16, 16)]
                u = m * (1.0 - _MOM) + b * _MOM
                us.append(u)
                ss = ss + u * u
            scale = _rsqrt(_hsum16(ss))
            for k in range(8):
                rv[r, pl.ds(k * 16, 16)] = us[k] * scale
            return 0

        lax.fori_loop(0, _Q, row, 0)
        pltpu.sync_copy(rv, upd_hbm.at[pl.ds(base + q * _Q, _Q)])

    plsc.subcore_barrier()
    p0 = w * _BW
    pltpu.sync_copy(idx_hbm.at[pl.ds(p0, _BW)], segi)
    pltpu.async_copy(winner_sp.at[segi], segw, sem).wait()
    pltpu.sync_copy(segw, winpos_hbm.at[pl.ds(p0, _BW)])


def _k2_body(lat_hbm, upd_hbm, idx_hbm, winpos_hbm, fb_hbm,
             out_hbm,
             segi, segw, cidx, cwpos, fbv, rows0, rows1, cbuf0, cbuf1,
             sem0, sem1, wsem0, wsem1):
    c = lax.axis_index("c")
    s = lax.axis_index("s")
    lo = c * _HALF_N

    r0 = lo + s * _SHARE
    _CH = 256
    ncp = _SHARE // _CH
    tail = _SHARE - ncp * _CH

    pltpu.async_copy(lat_hbm.at[pl.ds(r0, _CH)], cbuf0, sem0)
    for j in range(ncp):
        buf, sem, wsem = (cbuf0, sem0, wsem0) if j % 2 == 0 else \
                         (cbuf1, sem1, wsem1)
        nbuf, nsem, nwsem = (cbuf1, sem1, wsem1) if j % 2 == 0 else \
                            (cbuf0, sem0, wsem0)
        pltpu.make_async_copy(lat_hbm.at[pl.ds(r0 + j * _CH, _CH)],
                              buf, sem).wait()
        if j + 1 < ncp:
            if j >= 1:
                pltpu.make_async_copy(
                    nbuf, out_hbm.at[pl.ds(r0 + (j - 1) * _CH, _CH)],
                    nwsem).wait()
            pltpu.async_copy(lat_hbm.at[pl.ds(r0 + (j + 1) * _CH, _CH)],
                             nbuf, nsem)
        pltpu.async_copy(buf, out_hbm.at[pl.ds(r0 + j * _CH, _CH)], wsem)
    for j in (ncp - 2, ncp - 1):
        buf, wsem = (cbuf0, wsem0) if j % 2 == 0 else (cbuf1, wsem1)
        pltpu.make_async_copy(buf, out_hbm.at[pl.ds(r0 + j * _CH, _CH)],
                              wsem).wait()

    t0 = r0 + ncp * _CH
    pltpu.async_copy(lat_hbm.at[pl.ds(t0, tail)],
                     cbuf0.at[pl.ds(0, tail)], sem0).wait()
    pltpu.sync_copy(cbuf0.at[pl.ds(0, tail)], out_hbm.at[pl.ds(t0, tail)])

    @pl.when(s == _NS - 1)
    def _():
        r1 = lo + _NS * _SHARE
        rem = _HALF_N - _NS * _SHARE
        pltpu.async_copy(lat_hbm.at[pl.ds(r1, rem)],
                         cbuf1.at[pl.ds(0, rem)], sem1).wait()
        pltpu.sync_copy(cbuf1.at[pl.ds(0, rem)], out_hbm.at[pl.ds(r1, rem)])

    plsc.subcore_barrier()

    p0 = s * _SEG
    pltpu.sync_copy(idx_hbm.at[pl.ds(p0, _SEG)], segi)
    pltpu.sync_copy(winpos_hbm.at[pl.ds(p0, _SEG)], segw)
    pltpu.sync_copy(fb_hbm.at[c], fbv)
    zeros = jnp.zeros((16,), jnp.int32)
    jf = _shuf(fbv[pl.ds(0, 16)], zeros)
    wf = _shuf(fbv[pl.ds(0, 16)], zeros + 1)

    lane = _LANE()

    def fill(t, _):
        iv = segi[pl.ds(t * 16, 16)]
        wv = segw[pl.ds(t * 16, 16)]
        owned = jnp.logical_and(iv >= lo, iv < lo + _HALF_N)
        mn = jnp.where(owned, lane, 64)
        for k in (8, 4, 2, 1):
            mn = jnp.minimum(mn, _shuf(mn, lax.bitwise_xor(lane, k)))
        has = mn < 16
        perm = jnp.where(has, mn, 0)
        iv2 = jnp.where(has, _shuf(iv, perm), jf)
        wv2 = jnp.where(has, _shuf(wv, perm), wf)
        cidx[t // 8, pl.ds((t % 8) * 16, 16)] = jnp.where(owned, iv, iv2)
        cwpos[t // 8, pl.ds((t % 8) * 16, 16)] = jnp.where(owned, wv, wv2)
        return 0

    lax.fori_loop(0, _SEG // 16, fill, 0)

    nch = _SEG // 128
    pltpu.async_copy(upd_hbm.at[cwpos.at[0]], rows0, sem0)
    for j in range(nch):
        buf, sem = (rows0, sem0) if j % 2 == 0 else (rows1, sem1)
        nbuf, nsem = (rows1, sem1) if j % 2 == 0 else (rows0, sem0)
        pltpu.make_async_copy(upd_hbm.at[cwpos.at[j]], buf, sem).wait()
        if j + 1 < nch:
            pltpu.async_copy(upd_hbm.at[cwpos.at[j + 1]], nbuf, nsem)
        pltpu.sync_copy(buf, out_hbm.at[cidx.at[j]])


def _sc_call(inputs, body, out_type, scratch):
    mesh = plsc.VectorSubcoreMesh(core_axis_name="c", subcore_axis_name="s",
                                  num_cores=_NC, num_subcores=_NS)
    return pl.kernel(body, out_type=out_type, mesh=mesh,
                     scratch_types=scratch)(*inputs)


def _k3_body(t_ref, u_ref, cs_ref, out_ref):
    i = pl.program_id(0)
    t = t_ref[0, 0, :]
    oh = jnp.where(
        lax.broadcasted_iota(jnp.int32, (_C, t.shape[0]), 0) == t[None, :],
        1.0, 0.0)
    acc = jnp.dot(oh, u_ref[...], preferred_element_type=jnp.float32,
                  precision=lax.Precision.HIGHEST)

    @pl.when(i == 0)
    def _():
        out_ref[...] = cs_ref[...] + acc

    @pl.when(i > 0)
    def _():
        out_ref[...] = out_ref[...] + acc


def kernel(batch_samples, targets, idx, lat_memory, class_sums):
    pos = jnp.arange(_B, dtype=jnp.int32)

    fb_rows = []
    for h in range(_NC):
        in_half = jnp.logical_and(idx >= h * _HALF_N, idx < (h + 1) * _HALF_N)
        any_h = jnp.any(in_half)
        p = jnp.argmax(in_half).astype(jnp.int32)
        jf = jnp.where(any_h, idx[p], h * _HALF_N)
        fb_rows.append([jf, jnp.where(any_h, -1, _B + h * 8), p])
    fb = jnp.zeros((_NC, 16), jnp.int32)
    fb = fb.at[0, 0].set(fb_rows[0][0]).at[1, 0].set(fb_rows[1][0])

    upd, winpos = _sc_call(
        (batch_samples, idx, pos, lat_memory),
        _k1_body,
        [jax.ShapeDtypeStruct((_UPAD, _D), jnp.float32),
         jax.ShapeDtypeStruct((_B,), jnp.int32)],
        [
            pltpu.VMEM((_BW,), jnp.int32),
            pltpu.VMEM((_BW // 4, _D), jnp.float32),
            pltpu.VMEM((_BW // 4, _D), jnp.float32),
            pltpu.VMEM((_BW // 4, _D), jnp.float32),
            pltpu.VMEM((_BW // 4, _D), jnp.float32),
            pltpu.VMEM((_B,), jnp.int32),
            pltpu.VMEM((_B,), jnp.int32),
            pltpu.VMEM((_BW,), jnp.int32),
            pltpu.VMEM((_BW,), jnp.int32),
            pltpu.VMEM_SHARED((_N,), jnp.int32),
            pltpu.SemaphoreType.DMA,
            pltpu.SemaphoreType.DMA,
        ])

    wsel = jnp.stack([
        jnp.where(fb_rows[0][1] < 0, winpos[fb_rows[0][2]], fb_rows[0][1]),
        jnp.where(fb_rows[1][1] < 0, winpos[fb_rows[1][2]], fb_rows[1][1]),
    ]).astype(jnp.int32)
    fb = fb.at[0, 1].set(wsel[0]).at[1, 1].set(wsel[1])

    new_lat = _sc_call(
        (lat_memory, upd, idx, winpos, fb),
        _k2_body,
        jax.ShapeDtypeStruct((_N, _D), jnp.float32),
        [
            pltpu.VMEM((_SEG,), jnp.int32),
            pltpu.VMEM((_SEG,), jnp.int32),
            pltpu.VMEM((_SEG // 128, 128), jnp.int32),
            pltpu.VMEM((_SEG // 128, 128), jnp.int32),
            pltpu.VMEM((16,), jnp.int32),
            pltpu.VMEM((128, _D), jnp.float32),
            pltpu.VMEM((128, _D), jnp.float32),
            pltpu.VMEM((256, _D), jnp.float32),
            pltpu.VMEM((256, _D), jnp.float32),
            pltpu.SemaphoreType.DMA,
            pltpu.SemaphoreType.DMA,
            pltpu.SemaphoreType.DMA,
            pltpu.SemaphoreType.DMA,
        ])

    tchunk = 2048
    new_cs = pl.pallas_call(
        _k3_body,
        grid=(_B // tchunk,),
        in_specs=[
            pl.BlockSpec((1, 1, tchunk), lambda i: (i, 0, 0)),
            pl.BlockSpec((tchunk, _D), lambda i: (i, 0)),
            pl.BlockSpec((_C, _D), lambda i: (0, 0)),
        ],
        out_specs=pl.BlockSpec((_C, _D), lambda i: (0, 0)),
        out_shape=jax.ShapeDtypeStruct((_C, _D), jnp.float32),
    )(targets.reshape(_B // tchunk, 1, tchunk), upd, class_sums)

    return new_lat, new_cs


# speedup vs baseline: 1.0294x; 1.0046x over previous
import jax
import jax.numpy as jnp
from jax import lax
from jax.experimental import pallas as pl
from jax.experimental.pallas import tpu as pltpu
from jax.experimental.pallas import tpu_sc as plsc

_MOM = 0.9
_B = 16384
_D = 128
_N = 200000
_C = 100
_NC = 2
_NS = 16
_NW = _NC * _NS
_BW = _B // _NW
_SEG = _B // _NS
_HALF_N = _N // _NC
_SHARE = 6248
_UPAD = _B + _NC * 8

_GDN = lax.GatherDimensionNumbers(
    offset_dims=(), collapsed_slice_dims=(0,), start_index_map=(0,))


def _shuf(v, perm):
    return lax.gather(v, perm[:, None], _GDN, (1,),
                      mode=lax.GatherScatterMode.PROMISE_IN_BOUNDS)


_LANE = lambda: lax.broadcasted_iota(jnp.int32, (16,), 0)


def _hsum16(v):
    lane = _LANE()
    for k in (8, 4, 2, 1):
        v = v + _shuf(v, lax.bitwise_xor(lane, k))
    return v


def _rsqrt(s):
    i = lax.bitcast_convert_type(s, jnp.int32)
    i = jnp.int32(0x5F3759DF) - lax.shift_right_arithmetic(i, 1)
    y = lax.bitcast_convert_type(i, jnp.float32)
    for _ in range(3):
        y = y * (1.5 - 0.5 * s * y * y)
    return y


def _k1_body(batch_hbm, idx_hbm, pos_hbm, lat_hbm,
             upd_hbm, winpos_hbm,
             idxv, rowsv, bv, rowsv1, bv1, idxfull, posv, segi, segw,
             winner_sp, sem, sem1):
    c = lax.axis_index("c")
    s = lax.axis_index("s")
    w = c * _NS + s
    base = w * _BW

    @pl.when(s == 0)
    def _():
        pltpu.sync_copy(idx_hbm, idxfull)
        pltpu.sync_copy(pos_hbm, posv)
        pltpu.sync_copy(posv, winner_sp.at[idxfull])

    @pl.when(s == 1)
    def _():
        pltpu.async_copy(lat_hbm.at[pl.ds(c * _HALF_N, 8)],
                         rowsv.at[pl.ds(0, 8)], sem).wait()
        pltpu.sync_copy(rowsv.at[pl.ds(0, 8)],
                        upd_hbm.at[pl.ds(_B + c * 8, 8)])

    pltpu.sync_copy(idx_hbm.at[pl.ds(base, _BW)], idxv)
    _Q = _BW // 4

    def fire(q, rv, bb, sm):
        pltpu.async_copy(lat_hbm.at[idxv.at[pl.ds(q * _Q, _Q)]], rv, sm)
        pltpu.async_copy(batch_hbm.at[pl.ds(base + q * _Q, _Q)], bb, sm)

    fire(0, rowsv, bv, sem)
    for q in range(4):
        rv, bb, sm = (rowsv, bv, sem) if q % 2 == 0 else (rowsv1, bv1, sem1)
        nrv, nbb, nsm = (rowsv1, bv1, sem1) if q % 2 == 0 else (rowsv, bv, sem)
        pltpu.make_async_copy(lat_hbm.at[idxv.at[pl.ds(q * _Q, _Q)]],
                              rv, sm).wait()
        pltpu.make_async_copy(batch_hbm.at[pl.ds(base + q * _Q, _Q)],
                              bb, sm).wait()
        if q + 1 < 4:
            fire(q + 1, nrv, nbb, nsm)

        def row(r, _):
            ss = jnp.zeros((16,), jnp.float32)
            us = []
            for k in range(8):
                m = rv[r, pl.ds(k * 16, 16)]
                b = bb[r, pl.ds(k * 16, 16)]
                u = m * (1.0 - _MOM) + b * _MOM
                us.append(u)
                ss = ss + u * u
            scale = _rsqrt(_hsum16(ss))
            for k in range(8):
                rv[r, pl.ds(k * 16, 16)] = us[k] * scale
            return 0

        lax.fori_loop(0, _Q, row, 0)
        pltpu.sync_copy(rv, upd_hbm.at[pl.ds(base + q * _Q, _Q)])

    plsc.subcore_barrier()
    p0 = w * _BW
    pltpu.sync_copy(idx_hbm.at[pl.ds(p0, _BW)], segi)
    pltpu.async_copy(winner_sp.at[segi], segw, sem).wait()
    pltpu.sync_copy(segw, winpos_hbm.at[pl.ds(p0, _BW)])


def _k2_body(lat_hbm, upd_hbm, idx_hbm, winpos_hbm, fb_hbm,
             out_hbm,
             segi, segw, cidx, cwpos, fbv, rows0, rows1, cbuf0, cbuf1,
             sem0, sem1, wsem0, wsem1):
    c = lax.axis_index("c")
    s = lax.axis_index("s")
    lo = c * _HALF_N

    r0 = lo + s * _SHARE
    _CH = 256
    ncp = _SHARE // _CH
    tail = _SHARE - ncp * _CH

    pltpu.async_copy(lat_hbm.at[pl.ds(r0, _CH)], cbuf0, sem0)
    for j in range(ncp):
        buf, sem, wsem = (cbuf0, sem0, wsem0) if j % 2 == 0 else \
                         (cbuf1, sem1, wsem1)
        nbuf, nsem, nwsem = (cbuf1, sem1, wsem1) if j % 2 == 0 else \
                            (cbuf0, sem0, wsem0)
        pltpu.make_async_copy(lat_hbm.at[pl.ds(r0 + j * _CH, _CH)],
                              buf, sem).wait()
        if j + 1 < ncp:
            if j >= 1:
                pltpu.make_async_copy(
                    nbuf, out_hbm.at[pl.ds(r0 + (j - 1) * _CH, _CH)],
                    nwsem).wait()
            pltpu.async_copy(lat_hbm.at[pl.ds(r0 + (j + 1) * _CH, _CH)],
                             nbuf, nsem)
        pltpu.async_copy(buf, out_hbm.at[pl.ds(r0 + j * _CH, _CH)], wsem)
    for j in (ncp - 2, ncp - 1):
        buf, wsem = (cbuf0, wsem0) if j % 2 == 0 else (cbuf1, wsem1)
        pltpu.make_async_copy(buf, out_hbm.at[pl.ds(r0 + j * _CH, _CH)],
                              wsem).wait()

    t0 = r0 + ncp * _CH
    pltpu.async_copy(lat_hbm.at[pl.ds(t0, tail)],
                     cbuf0.at[pl.ds(0, tail)], sem0).wait()
    pltpu.sync_copy(cbuf0.at[pl.ds(0, tail)], out_hbm.at[pl.ds(t0, tail)])

    @pl.when(s == _NS - 1)
    def _():
        r1 = lo + _NS * _SHARE
        rem = _HALF_N - _NS * _SHARE
        pltpu.async_copy(lat_hbm.at[pl.ds(r1, rem)],
                         cbuf1.at[pl.ds(0, rem)], sem1).wait()
        pltpu.sync_copy(cbuf1.at[pl.ds(0, rem)], out_hbm.at[pl.ds(r1, rem)])

    plsc.subcore_barrier()

    p0 = s * _SEG
    pltpu.sync_copy(idx_hbm.at[pl.ds(p0, _SEG)], segi)
    pltpu.sync_copy(winpos_hbm.at[pl.ds(p0, _SEG)], segw)
    pltpu.sync_copy(fb_hbm.at[c], fbv)
    zeros = jnp.zeros((16,), jnp.int32)
    jf = _shuf(fbv[pl.ds(0, 16)], zeros)
    wf = _shuf(fbv[pl.ds(0, 16)], zeros + 1)

    lane = _LANE()

    def fill(t, _):
        iv = segi[pl.ds(t * 16, 16)]
        wv = segw[pl.ds(t * 16, 16)]
        owned = jnp.logical_and(iv >= lo, iv < lo + _HALF_N)
        mn = jnp.where(owned, lane, 64)
        for k in (8, 4, 2, 1):
            mn = jnp.minimum(mn, _shuf(mn, lax.bitwise_xor(lane, k)))
        has = mn < 16
        perm = jnp.where(has, mn, 0)
        iv2 = jnp.where(has, _shuf(iv, perm), jf)
        wv2 = jnp.where(has, _shuf(wv, perm), wf)
        cidx[t // 8, pl.ds((t % 8) * 16, 16)] = jnp.where(owned, iv, iv2)
        cwpos[t // 8, pl.ds((t % 8) * 16, 16)] = jnp.where(owned, wv, wv2)
        return 0

    lax.fori_loop(0, _SEG // 16, fill, 0)

    nch = _SEG // 128
    pltpu.async_copy(upd_hbm.at[cwpos.at[0]], rows0, sem0)
    for j in range(nch):
        buf, sem = (rows0, sem0) if j % 2 == 0 else (rows1, sem1)
        nbuf, nsem = (rows1, sem1) if j % 2 == 0 else (rows0, sem0)
        pltpu.make_async_copy(upd_hbm.at[cwpos.at[j]], buf, sem).wait()
        if j + 1 < nch:
            pltpu.async_copy(upd_hbm.at[cwpos.at[j + 1]], nbuf, nsem)
        pltpu.sync_copy(buf, out_hbm.at[cidx.at[j]])


def _sc_call(inputs, body, out_type, scratch):
    mesh = plsc.VectorSubcoreMesh(core_axis_name="c", subcore_axis_name="s",
                                  num_cores=_NC, num_subcores=_NS)
    return pl.kernel(body, out_type=out_type, mesh=mesh,
                     scratch_types=scratch)(*inputs)


def _k3_body(t_ref, u_ref, cs_ref, out_ref):
    i = pl.program_id(0)
    t = t_ref[0, 0, :]
    oh = jnp.where(
        lax.broadcasted_iota(jnp.int32, (_C, t.shape[0]), 0) == t[None, :],
        1.0, 0.0)
    acc = jnp.dot(oh, u_ref[...], preferred_element_type=jnp.float32,
                  precision=lax.Precision.HIGHEST)

    @pl.when(i == 0)
    def _():
        out_ref[...] = cs_ref[...] + acc

    @pl.when(i > 0)
    def _():
        out_ref[...] = out_ref[...] + acc


def kernel(batch_samples, targets, idx, lat_memory, class_sums):
    pos = jnp.arange(_B, dtype=jnp.int32)

    fb_rows = []
    for h in range(_NC):
        in_half = jnp.logical_and(idx >= h * _HALF_N, idx < (h + 1) * _HALF_N)
        any_h = jnp.any(in_half)
        p = jnp.argmax(in_half).astype(jnp.int32)
        jf = jnp.where(any_h, idx[p], h * _HALF_N)
        fb_rows.append([jf, jnp.where(any_h, -1, _B + h * 8), p])
    fb = jnp.zeros((_NC, 16), jnp.int32)
    fb = fb.at[0, 0].set(fb_rows[0][0]).at[1, 0].set(fb_rows[1][0])

    upd, winpos = _sc_call(
        (batch_samples, idx, pos, lat_memory),
        _k1_body,
        [jax.ShapeDtypeStruct((_UPAD, _D), jnp.float32),
         jax.ShapeDtypeStruct((_B,), jnp.int32)],
        [
            pltpu.VMEM((_BW,), jnp.int32),
            pltpu.VMEM((_BW // 4, _D), jnp.float32),
            pltpu.VMEM((_BW // 4, _D), jnp.float32),
            pltpu.VMEM((_BW // 4, _D), jnp.float32),
            pltpu.VMEM((_BW // 4, _D), jnp.float32),
            pltpu.VMEM((_B,), jnp.int32),
            pltpu.VMEM((_B,), jnp.int32),
            pltpu.VMEM((_BW,), jnp.int32),
            pltpu.VMEM((_BW,), jnp.int32),
            pltpu.VMEM_SHARED((_N,), jnp.int32),
            pltpu.SemaphoreType.DMA,
            pltpu.SemaphoreType.DMA,
        ])

    wsel = jnp.stack([
        jnp.where(fb_rows[0][1] < 0, winpos[fb_rows[0][2]], fb_rows[0][1]),
        jnp.where(fb_rows[1][1] < 0, winpos[fb_rows[1][2]], fb_rows[1][1]),
    ]).astype(jnp.int32)
    fb = fb.at[0, 1].set(wsel[0]).at[1, 1].set(wsel[1])

    new_lat = _sc_call(
        (lat_memory, upd, idx, winpos, fb),
        _k2_body,
        jax.ShapeDtypeStruct((_N, _D), jnp.float32),
        [
            pltpu.VMEM((_SEG,), jnp.int32),
            pltpu.VMEM((_SEG,), jnp.int32),
            pltpu.VMEM((_SEG // 128, 128), jnp.int32),
            pltpu.VMEM((_SEG // 128, 128), jnp.int32),
            pltpu.VMEM((16,), jnp.int32),
            pltpu.VMEM((128, _D), jnp.float32),
            pltpu.VMEM((128, _D), jnp.float32),
            pltpu.VMEM((256, _D), jnp.float32),
            pltpu.VMEM((256, _D), jnp.float32),
            pltpu.SemaphoreType.DMA,
            pltpu.SemaphoreType.DMA,
            pltpu.SemaphoreType.DMA,
            pltpu.SemaphoreType.DMA,
        ])

    tchunk = 2048
    new_cs = pl.pallas_call(
        _k3_body,
        grid=(_B // tchunk,),
        in_specs=[
            pl.BlockSpec((1, 1, tchunk), lambda i: (i, 0, 0)),
            pl.BlockSpec((tchunk, _D), lambda i: (i, 0)),
            pl.BlockSpec((_C, _D), lambda i: (0, 0)),
        ],
        out_specs=pl.BlockSpec((_C, _D), lambda i: (0, 0)),
        out_shape=jax.ShapeDtypeStruct((_C, _D), jnp.float32),
    )(targets.reshape(_B // tchunk, 1, tchunk), upd, class_sums)

    return new_lat, new_cs
